# Initial kernel scaffold; baseline (speedup 1.0000x reference)
import jax
import jax.numpy as jnp
from jax.experimental import pallas as pl


def kernel(x, embeddings):
    raise NotImplementedError("write your pallas kernel here")


# speedup vs baseline: 7.1989x; 7.1989x over previous
import jax
import jax.numpy as jnp
from jax import lax
from jax.experimental import pallas as pl

_BB = 128


def _body(xt_ref, emb_ref, out_ref):
    t, bb = xt_ref.shape
    d = emb_ref.shape[1]
    maskt = xt_ref[...] == 0
    erows = emb_ref[1:t + 1, :]
    e0b = jnp.broadcast_to(emb_ref[0:1, :], (t, d))
    for b in range(bb):
        mcol = jnp.broadcast_to(maskt[:, b:b + 1], (t, d))
        out_ref[b] = jnp.where(mcol, e0b, erows)


def kernel(x, embeddings):
    b, t = x.shape
    v, d = embeddings.shape
    out = pl.pallas_call(
        _body,
        grid=(b // _BB,),
        in_specs=[
            pl.BlockSpec((t, _BB), lambda i: (0, i)),
            pl.BlockSpec((v, d), lambda i: (0, 0)),
        ],
        out_specs=pl.BlockSpec((_BB, t, d), lambda i: (i, 0, 0)),
        out_shape=jax.ShapeDtypeStruct((b, t, d), jnp.float32),
    )(x.T, embeddings)
    return out
